# baseline (device time: 104506 ns/iter reference)
import jax
import jax.numpy as jnp
from jax import lax
from jax.experimental import pallas as pl
from jax.experimental.pallas import tpu as pltpu

T = 4096
D = 1024


def kernel(x, dest):
    order = jnp.argsort(dest, stable=True).astype(jnp.int32)
    xb = x.astype(jnp.bfloat16).reshape(T, 8, 128)
    n0 = jnp.sum(dest == 0).astype(jnp.int32).reshape((1,))

    def body(sc_ref, order_ref, xb_ref, out_ref, send_sem, recv_sem, copy_sem):
        me = lax.axis_index("y")
        my_x = lax.axis_index("x")
        peer = (my_x, 1 - me)

        n0v = sc_ref[0]
        n_keep = jnp.where(me == 0, n0v, T - n0v)
        n_send = T - n_keep
        n_recv = T - n_keep
        keep_src = jnp.where(me == 0, 0, n0v)
        send_src = jnp.where(me == 0, n0v, 0)
        keep_dst = jnp.where(me == 0, 0, T - n_keep)
        send_dst = jnp.where(me == 0, 0, T - n_send)

        barrier = pltpu.get_barrier_semaphore()
        pl.semaphore_signal(
            barrier, inc=1, device_id=peer,
            device_id_type=pl.DeviceIdType.MESH,
        )
        pl.semaphore_wait(barrier, 1)

        def send_body(j, carry):
            r = order_ref[send_src + j]
            rdma = pltpu.make_async_remote_copy(
                src_ref=xb_ref.at[pl.ds(r, 1)],
                dst_ref=out_ref.at[pl.ds(send_dst + j, 1)],
                send_sem=send_sem,
                recv_sem=recv_sem,
                device_id=peer,
                device_id_type=pl.DeviceIdType.MESH,
            )
            rdma.start()
            return carry

        lax.fori_loop(0, n_send, send_body, 0)

        def keep_body(j, carry):
            r = order_ref[keep_src + j]
            cp = pltpu.make_async_copy(
                xb_ref.at[pl.ds(r, 1)],
                out_ref.at[pl.ds(keep_dst + j, 1)],
                copy_sem,
            )
            cp.start()
            return carry

        lax.fori_loop(0, n_keep, keep_body, 0)

        def wait_keep(j, carry):
            pltpu.make_async_copy(
                xb_ref.at[pl.ds(0, 1)], out_ref.at[pl.ds(0, 1)], copy_sem
            ).wait()
            return carry

        lax.fori_loop(0, n_keep, wait_keep, 0)

        def wait_rdma(j, carry):
            rdma = pltpu.make_async_remote_copy(
                src_ref=xb_ref.at[pl.ds(0, 1)],
                dst_ref=out_ref.at[pl.ds(0, 1)],
                send_sem=send_sem,
                recv_sem=recv_sem,
                device_id=peer,
                device_id_type=pl.DeviceIdType.MESH,
            )
            rdma.wait_recv()
            return carry

        lax.fori_loop(0, n_recv, wait_rdma, 0)

        def wait_send(j, carry):
            rdma = pltpu.make_async_remote_copy(
                src_ref=xb_ref.at[pl.ds(0, 1)],
                dst_ref=out_ref.at[pl.ds(0, 1)],
                send_sem=send_sem,
                recv_sem=recv_sem,
                device_id=peer,
                device_id_type=pl.DeviceIdType.MESH,
            )
            rdma.wait_send()
            return carry

        lax.fori_loop(0, n_send, wait_send, 0)

    out = pl.pallas_call(
        body,
        out_shape=jax.ShapeDtypeStruct((T, 8, 128), jnp.bfloat16),
        in_specs=[
            pl.BlockSpec(memory_space=pltpu.SMEM),
            pl.BlockSpec(memory_space=pltpu.SMEM),
            pl.BlockSpec(memory_space=pltpu.VMEM),
        ],
        out_specs=pl.BlockSpec(memory_space=pltpu.VMEM),
        scratch_shapes=[
            pltpu.SemaphoreType.DMA,
            pltpu.SemaphoreType.DMA,
            pltpu.SemaphoreType.DMA,
        ],
        compiler_params=pltpu.CompilerParams(collective_id=0),
    )(n0, order, xb)
    return out.reshape(T, D)


# device time: 85543 ns/iter; 1.2217x vs baseline; 1.2217x over previous
import jax
import jax.numpy as jnp
from jax import lax
from jax.experimental import pallas as pl
from jax.experimental.pallas import tpu as pltpu

T = 4096
D = 1024
CHUNK = 512
MAX_CHUNKS = T // CHUNK


def kernel(x, dest):
    order = jnp.argsort(dest, stable=True).astype(jnp.int32)
    x3 = x.reshape(T, 8, 128)
    n0 = jnp.sum(dest == 0).astype(jnp.int32).reshape((1,))

    def body(sc_ref, order_ref, x_ref, out_ref, sbuf, send_sems, recv_sems):
        me = lax.axis_index("y")
        my_x = lax.axis_index("x")
        peer = (my_x, 1 - me)

        n0v = sc_ref[0]
        n_keep = jnp.where(me == 0, n0v, T - n0v)
        n_send = T - n_keep
        n_recv = T - n_keep
        keep_src = jnp.where(me == 0, 0, n0v)
        send_src = jnp.where(me == 0, n0v, 0)
        keep_dst = jnp.where(me == 0, 0, T - n_keep)
        send_dst = jnp.where(me == 0, 0, T - n_send)
        nc_send = (n_send + CHUNK - 1) // CHUNK
        nc_recv = (n_recv + CHUNK - 1) // CHUNK

        barrier = pltpu.get_barrier_semaphore()
        pl.semaphore_signal(
            barrier, inc=1, device_id=peer,
            device_id_type=pl.DeviceIdType.MESH,
        )
        pl.semaphore_wait(barrier, 1)

        for c in range(MAX_CHUNKS):
            @pl.when(c < nc_send)
            def _(c=c):
                end = jnp.minimum((c + 1) * CHUNK, n_send)

                def g(j, carry):
                    r = order_ref[send_src + j]
                    sbuf[pl.ds(j, 1), :, :] = (
                        x_ref[pl.ds(r, 1), :, :].astype(jnp.bfloat16)
                    )
                    return carry

                lax.fori_loop(c * CHUNK, end, g, 0)

                off = jnp.maximum(0, jnp.minimum(c * CHUNK, n_send - CHUNK))
                rdma = pltpu.make_async_remote_copy(
                    src_ref=sbuf.at[pl.ds(off, CHUNK)],
                    dst_ref=out_ref.at[pl.ds(send_dst + off, CHUNK)],
                    send_sem=send_sems.at[c],
                    recv_sem=recv_sems.at[c],
                    device_id=peer,
                    device_id_type=pl.DeviceIdType.MESH,
                )
                rdma.start()

        def k(j, carry):
            r = order_ref[keep_src + j]
            out_ref[pl.ds(keep_dst + j, 1), :, :] = (
                x_ref[pl.ds(r, 1), :, :].astype(jnp.bfloat16)
            )
            return carry

        lax.fori_loop(0, n_keep, k, 0)

        for c in range(MAX_CHUNKS):
            @pl.when(c < nc_recv)
            def _(c=c):
                rdma = pltpu.make_async_remote_copy(
                    src_ref=sbuf.at[pl.ds(0, CHUNK)],
                    dst_ref=out_ref.at[pl.ds(0, CHUNK)],
                    send_sem=send_sems.at[c],
                    recv_sem=recv_sems.at[c],
                    device_id=peer,
                    device_id_type=pl.DeviceIdType.MESH,
                )
                rdma.wait_recv()

        for c in range(MAX_CHUNKS):
            @pl.when(c < nc_send)
            def _(c=c):
                rdma = pltpu.make_async_remote_copy(
                    src_ref=sbuf.at[pl.ds(0, CHUNK)],
                    dst_ref=out_ref.at[pl.ds(0, CHUNK)],
                    send_sem=send_sems.at[c],
                    recv_sem=recv_sems.at[c],
                    device_id=peer,
                    device_id_type=pl.DeviceIdType.MESH,
                )
                rdma.wait_send()

    out = pl.pallas_call(
        body,
        out_shape=jax.ShapeDtypeStruct((T, 8, 128), jnp.bfloat16),
        in_specs=[
            pl.BlockSpec(memory_space=pltpu.SMEM),
            pl.BlockSpec(memory_space=pltpu.SMEM),
            pl.BlockSpec(memory_space=pltpu.VMEM),
        ],
        out_specs=pl.BlockSpec(memory_space=pltpu.VMEM),
        scratch_shapes=[
            pltpu.VMEM((T, 8, 128), jnp.bfloat16),
            pltpu.SemaphoreType.DMA((MAX_CHUNKS,)),
            pltpu.SemaphoreType.DMA((MAX_CHUNKS,)),
        ],
        compiler_params=pltpu.CompilerParams(collective_id=0),
    )(n0, order, x3)
    return out.reshape(T, D)


# device time: 75394 ns/iter; 1.3861x vs baseline; 1.1346x over previous
import jax
import jax.numpy as jnp
from jax import lax
from jax.experimental import pallas as pl
from jax.experimental.pallas import tpu as pltpu

T = 4096
D = 1024
CHUNK = 512
MAX_CHUNKS = T // CHUNK


def kernel(x, dest):
    order = jnp.argsort(dest, stable=True).astype(jnp.int32)
    x3 = x.astype(jnp.bfloat16).reshape(T, 8, 128)
    n0 = jnp.sum(dest == 0).astype(jnp.int32).reshape((1,))

    def body(sc_ref, order_ref, x_ref, out_ref, sbuf, send_sems, recv_sems):
        me = lax.axis_index("y")
        my_x = lax.axis_index("x")
        peer = (my_x, 1 - me)

        n0v = sc_ref[0]
        n_keep = jnp.where(me == 0, n0v, T - n0v)
        n_send = T - n_keep
        n_recv = T - n_keep
        keep_src = jnp.where(me == 0, 0, n0v)
        send_src = jnp.where(me == 0, n0v, 0)
        keep_dst = jnp.where(me == 0, 0, T - n_keep)
        send_dst = jnp.where(me == 0, 0, T - n_send)
        nc_send = (n_send + CHUNK - 1) // CHUNK
        nc_recv = (n_recv + CHUNK - 1) // CHUNK

        barrier = pltpu.get_barrier_semaphore()
        pl.semaphore_signal(
            barrier, inc=1, device_id=peer,
            device_id_type=pl.DeviceIdType.MESH,
        )
        pl.semaphore_wait(barrier, 1)

        for c in range(MAX_CHUNKS):
            @pl.when(c < nc_send)
            def _(c=c):
                end = jnp.minimum((c + 1) * CHUNK, n_send)

                def g(j, carry):
                    r = order_ref[send_src + j]
                    sbuf[pl.ds(j, 1), :, :] = x_ref[pl.ds(r, 1), :, :]
                    return carry

                lax.fori_loop(c * CHUNK, end, g, 0)

                off = jnp.maximum(0, jnp.minimum(c * CHUNK, n_send - CHUNK))
                rdma = pltpu.make_async_remote_copy(
                    src_ref=sbuf.at[pl.ds(off, CHUNK)],
                    dst_ref=out_ref.at[pl.ds(send_dst + off, CHUNK)],
                    send_sem=send_sems.at[c],
                    recv_sem=recv_sems.at[c],
                    device_id=peer,
                    device_id_type=pl.DeviceIdType.MESH,
                )
                rdma.start()

        def k(j, carry):
            r = order_ref[keep_src + j]
            out_ref[pl.ds(keep_dst + j, 1), :, :] = x_ref[pl.ds(r, 1), :, :]
            return carry

        lax.fori_loop(0, n_keep, k, 0)

        for c in range(MAX_CHUNKS):
            @pl.when(c < nc_recv)
            def _(c=c):
                rdma = pltpu.make_async_remote_copy(
                    src_ref=sbuf.at[pl.ds(0, CHUNK)],
                    dst_ref=out_ref.at[pl.ds(0, CHUNK)],
                    send_sem=send_sems.at[c],
                    recv_sem=recv_sems.at[c],
                    device_id=peer,
                    device_id_type=pl.DeviceIdType.MESH,
                )
                rdma.wait_recv()

        for c in range(MAX_CHUNKS):
            @pl.when(c < nc_send)
            def _(c=c):
                rdma = pltpu.make_async_remote_copy(
                    src_ref=sbuf.at[pl.ds(0, CHUNK)],
                    dst_ref=out_ref.at[pl.ds(0, CHUNK)],
                    send_sem=send_sems.at[c],
                    recv_sem=recv_sems.at[c],
                    device_id=peer,
                    device_id_type=pl.DeviceIdType.MESH,
                )
                rdma.wait_send()

    out = pl.pallas_call(
        body,
        out_shape=jax.ShapeDtypeStruct((T, 8, 128), jnp.bfloat16),
        in_specs=[
            pl.BlockSpec(memory_space=pltpu.SMEM),
            pl.BlockSpec(memory_space=pltpu.SMEM),
            pl.BlockSpec(memory_space=pltpu.VMEM),
        ],
        out_specs=pl.BlockSpec(memory_space=pltpu.VMEM),
        scratch_shapes=[
            pltpu.VMEM((T, 8, 128), jnp.bfloat16),
            pltpu.SemaphoreType.DMA((MAX_CHUNKS,)),
            pltpu.SemaphoreType.DMA((MAX_CHUNKS,)),
        ],
        compiler_params=pltpu.CompilerParams(collective_id=0),
    )(n0, order, x3)
    return out.reshape(T, D)
